# Initial kernel scaffold; baseline (speedup 1.0000x reference)
import jax
import jax.numpy as jnp
from jax.experimental import pallas as pl


def kernel(input, table):
    raise NotImplementedError("write your pallas kernel here")


# speedup vs baseline: 2.6495x; 2.6495x over previous
import functools

import jax
import jax.numpy as jnp
from jax import lax
from jax.experimental import pallas as pl
from jax.experimental.pallas import tpu as pltpu
from jax.experimental.pallas import tpu_sc as plsc

D = 32
B = 16384
H = 50
NC, NS = 2, 16
NW = NC * NS
CHUNK = 128
NCHUNK = B // CHUNK
L = 16

_mesh = plsc.VectorSubcoreMesh(core_axis_name="c", subcore_axis_name="s")


@functools.partial(
    pl.kernel,
    mesh=_mesh,
    compiler_params=pltpu.CompilerParams(use_tc_tiling_on_sc=False),
    out_type=jax.ShapeDtypeStruct((H, D), jnp.float32),
    scratch_types=[
        pltpu.VMEM((NCHUNK, CHUNK), jnp.int32),
        pltpu.VMEM((2, CHUNK, D), jnp.float32),
        pltpu.VMEM((D,), jnp.float32),
        pltpu.SemaphoreType.DMA((2,)),
    ],
)
def _cbow_sc(idx_hbm, table_hbm, out_hbm, idx_v, rows_v, st_v, sems):
    wid = lax.axis_index("s") * NC + lax.axis_index("c")

    def gather(c, p):
        return pltpu.make_async_copy(
            table_hbm.at[idx_v.at[c]], rows_v.at[p], sems.at[p]
        )

    def accum_chunk(buf, accs):
        def row_body(r, a):
            a0, a1, a2, a3, a4, a5, a6, a7 = a
            base = r * 4
            a0 = a0 + buf[base, pl.ds(0, L)]
            a1 = a1 + buf[base, pl.ds(L, L)]
            a2 = a2 + buf[base + 1, pl.ds(0, L)]
            a3 = a3 + buf[base + 1, pl.ds(L, L)]
            a4 = a4 + buf[base + 2, pl.ds(0, L)]
            a5 = a5 + buf[base + 2, pl.ds(L, L)]
            a6 = a6 + buf[base + 3, pl.ds(0, L)]
            a7 = a7 + buf[base + 3, pl.ds(L, L)]
            return (a0, a1, a2, a3, a4, a5, a6, a7)

        return lax.fori_loop(0, CHUNK // 4, row_body, accs)

    def process(h):
        pltpu.sync_copy(idx_hbm.at[h], idx_v)
        gather(0, 0).start()
        zero = jnp.zeros((L,), jnp.float32)

        def g_body(g, accs):
            c0 = g * 2
            gather(c0 + 1, 1).start()
            gather(0, 0).wait()
            accs = accum_chunk(rows_v.at[0], accs)

            @pl.when(c0 + 2 < NCHUNK)
            def _():
                gather(c0 + 2, 0).start()

            gather(0, 1).wait()
            return accum_chunk(rows_v.at[1], accs)

        accs = lax.fori_loop(0, NCHUNK // 2, g_body, (zero,) * 8)
        scale = jnp.float32(1.0 / B)
        lo = ((accs[0] + accs[2]) + (accs[4] + accs[6])) * scale
        hi = ((accs[1] + accs[3]) + (accs[5] + accs[7])) * scale
        st_v[pl.ds(0, L)] = lo
        st_v[pl.ds(L, L)] = hi
        pltpu.sync_copy(st_v, out_hbm.at[h])

    process(wid)

    @pl.when(wid + NW < H)
    def _():
        process(wid + NW)


def kernel(input, table):
    idx = jnp.transpose(input).astype(jnp.int32).reshape(H, NCHUNK, CHUNK)
    return _cbow_sc(idx, table)


# speedup vs baseline: 2.6801x; 1.0115x over previous
import functools

import jax
import jax.numpy as jnp
from jax import lax
from jax.experimental import pallas as pl
from jax.experimental.pallas import tpu as pltpu
from jax.experimental.pallas import tpu_sc as plsc

D = 32
B = 16384
H = 50
NC, NS = 2, 16
HPC = H // NC
BPT = B // NS
CHUNK = 128
NCHUNK = BPT // CHUNK
L = 16
ACC_ROWS = 32

_mesh = plsc.VectorSubcoreMesh(core_axis_name="c", subcore_axis_name="s")


@functools.partial(
    pl.kernel,
    mesh=_mesh,
    compiler_params=pltpu.CompilerParams(
        use_tc_tiling_on_sc=False, needs_layout_passes=False
    ),
    out_type=jax.ShapeDtypeStruct((H, D), jnp.float32),
    scratch_types=[
        pltpu.VMEM((BPT, H), jnp.int32),
        pltpu.VMEM((BPT,), jnp.int32),
        pltpu.VMEM((2, CHUNK, D), jnp.float32),
        pltpu.VMEM((ACC_ROWS, D), jnp.float32),
        pltpu.VMEM((ACC_ROWS,), jnp.int32),
        pltpu.VMEM((D,), jnp.float32),
        pltpu.VMEM_SHARED((ACC_ROWS, D), jnp.float32),
        pltpu.SemaphoreType.DMA((2,)),
    ],
)
def _cbow_sc(idx_hbm, table_hbm, out_hbm, idxblk_v, list_v, rows_v, acc_v,
             rowids_v, st_v, shared_acc, sems):
    c = lax.axis_index("c")
    s = lax.axis_index("s")
    lanes = jnp.arange(L, dtype=jnp.int32)

    def gather(k, p):
        return pltpu.make_async_copy(
            table_hbm.at[list_v.at[pl.ds(k * CHUNK, CHUNK)]],
            rows_v.at[p],
            sems.at[p],
        )

    def accum_chunk(buf, accs):
        def row_body(r, a):
            a0, a1, a2, a3, a4, a5, a6, a7 = a
            base = r * 4
            a0 = a0 + buf[base, pl.ds(0, L)]
            a1 = a1 + buf[base, pl.ds(L, L)]
            a2 = a2 + buf[base + 1, pl.ds(0, L)]
            a3 = a3 + buf[base + 1, pl.ds(L, L)]
            a4 = a4 + buf[base + 2, pl.ds(0, L)]
            a5 = a5 + buf[base + 2, pl.ds(L, L)]
            a6 = a6 + buf[base + 3, pl.ds(0, L)]
            a7 = a7 + buf[base + 3, pl.ds(L, L)]
            return (a0, a1, a2, a3, a4, a5, a6, a7)

        return lax.fori_loop(0, CHUNK // 4, row_body, accs)

    pltpu.sync_copy(idx_hbm.at[pl.ds(s * BPT, BPT)], idxblk_v)

    rowids_v[pl.ds(0, L)] = lanes
    rowids_v[pl.ds(L, L)] = lanes + L

    def j_body(j, _):
        h = 2 * j + c

        def compact_body(v, _):
            rows = lanes + v * L
            cols = jnp.broadcast_to(h, (L,)).astype(jnp.int32)
            list_v[pl.ds(v * L, L)] = plsc.load_gather(idxblk_v, [rows, cols])
            return 0

        lax.fori_loop(0, BPT // L, compact_body, 0)

        gather(0, 0).start()
        zero = jnp.zeros((L,), jnp.float32)

        def g_body(g, accs):
            k0 = g * 2
            gather(k0 + 1, 1).start()
            gather(0, 0).wait()
            accs = accum_chunk(rows_v.at[0], accs)

            @pl.when(k0 + 2 < NCHUNK)
            def _():
                gather(k0 + 2, 0).start()

            gather(0, 1).wait()
            return accum_chunk(rows_v.at[1], accs)

        accs = lax.fori_loop(0, NCHUNK // 2, g_body, (zero,) * 8)
        acc_v[j, pl.ds(0, L)] = (accs[0] + accs[2]) + (accs[4] + accs[6])
        acc_v[j, pl.ds(L, L)] = (accs[1] + accs[3]) + (accs[5] + accs[7])
        return 0

    lax.fori_loop(0, HPC, j_body, 0)

    @pl.when(s == 0)
    def _():
        pltpu.sync_copy(acc_v, shared_acc)

    plsc.subcore_barrier()

    @pl.when(s != 0)
    def _():
        pltpu.sync_copy(acc_v, shared_acc.at[rowids_v], add=True)

    plsc.subcore_barrier()

    scale = jnp.float32(1.0 / B)

    def writeout(hl):
        pltpu.sync_copy(shared_acc.at[hl], st_v)
        st_v[pl.ds(0, L)] = st_v[pl.ds(0, L)] * scale
        st_v[pl.ds(L, L)] = st_v[pl.ds(L, L)] * scale
        pltpu.sync_copy(st_v, out_hbm.at[2 * hl + c])

    writeout(s)

    @pl.when(s + NS < HPC)
    def _():
        writeout(s + NS)


def kernel(input, table):
    return _cbow_sc(input.astype(jnp.int32), table)
